# baseline (speedup 1.0000x reference)
import functools

import jax
import jax.numpy as jnp
from jax.experimental import pallas as pl
from jax.experimental.pallas import tpu as pltpu

RF_CENTERS = 96.5
TH_C = 0.6
IMG_H = 193.0
TH2 = 12.0 * IMG_H
N = 128
K = 1024


def _body(lx_ref, ly_ref, c0_ref, c1_ref, lab_ref, out_ref, min2_ref):
    labf = lab_ref[...]
    px = labf[:, 0:1] - RF_CENTERS
    py = labf[:, 1:2] - RF_CENTERS
    pp = px * px + py * py
    ones_p = jnp.ones((K, 1), jnp.float32)
    p4 = jnp.concatenate([px * -2.0, py * -2.0, pp, ones_p], axis=1)

    p4_chunks = [jax.lax.slice(p4, (c * N, 0), ((c + 1) * N, 4))
                 for c in range(K // N)]

    def row(r, _):
        qxr = lx_ref[pl.ds(r, 1), :] * 10.0
        qyr = ly_ref[pl.ds(r, 1), :] * 10.0
        qqr = qxr * qxr + qyr * qyr
        q4 = jnp.concatenate(
            [qxr, qyr, jnp.ones((1, N), jnp.float32), qqr], axis=0)
        acc = None
        for p4_c in p4_chunks:
            d = jax.lax.dot_general(
                p4_c, q4, (((1,), (0,)), ((), ())),
                preferred_element_type=jnp.float32)
            acc = d if acc is None else jnp.minimum(acc, d)
        min2_ref[pl.ds(r, 1), :] = jnp.min(acc, axis=0, keepdims=True)
        return 0

    jax.lax.fori_loop(0, N, row, 0, unroll=4)

    min2 = min2_ref[...]
    c0 = c0_ref[...]
    c1 = c1_ref[...]

    near = min2 < TH2
    far = min2 > TH2
    pos = c1 > TH_C
    neg = c0 > TH_C
    tp = pos & near
    fp = pos & far
    tn = neg & far
    fn = neg & near

    mx = jnp.maximum(c0, c1)
    lse = mx + jnp.log(jnp.exp(c0 - mx) + jnp.exp(c1 - mx))
    v0 = lse - c0
    v1 = lse - c1

    ii = jax.lax.broadcasted_iota(jnp.int32, (N, N), 0)
    jj = jax.lax.broadcasted_iota(jnp.int32, (N, N), 1)
    tri_incl = (ii <= jj).astype(jnp.float32)
    tri_strict = (jj < ii).astype(jnp.float32)

    def mask_stats(mask):
        m32 = mask.astype(jnp.float32)
        csum = jax.lax.dot_general(
            m32, tri_incl, (((1,), (0,)), ((), ())),
            preferred_element_type=jnp.float32)
        tot = csum[:, N - 1:N]
        pref = jax.lax.dot_general(
            tri_strict, tot, (((1,), (0,)), ((), ())),
            preferred_element_type=jnp.float32)
        rank_incl = pref + csum
        count = pref[N - 1, 0] + tot[N - 1, 0]
        return m32, rank_incl, count

    tp32, tp_rank, tp_n = mask_stats(tp)
    fp32, fp_rank, fp_n = mask_stats(fp)
    tn32, tn_rank, tn_n = mask_stats(tn)
    fn32, fn_rank, fn_n = mask_stats(fn)

    min_n = jnp.minimum(jnp.minimum(tp_n, fp_n), jnp.minimum(tn_n, fn_n))
    min_n = jnp.where(min_n == 0.0, 10.0, min_n)

    def term(m32, rank_incl, count, v):
        incl = (m32 > 0.0) & (rank_incl <= min_n)
        s = jnp.sum(jnp.where(incl, v, 0.0))
        n = jnp.minimum(count, min_n)
        return jnp.where(count > 0.0, s / jnp.maximum(n, 1.0), 0.0)

    loss = (term(tp32, tp_rank, tp_n, v1)
            + term(fp32, fp_rank, fp_n, v0)
            + term(tn32, tn_rank, tn_n, v0)
            + term(fn32, fn_rank, fn_n, v1))
    out_ref[0, 0] = loss


@jax.jit
def kernel(outputs_l, outputs_c, labels):
    lx = outputs_l[0, 0]
    ly = outputs_l[0, 1]
    c0 = outputs_c[0, 0]
    c1 = outputs_c[0, 1]
    labf = labels[0].astype(jnp.float32)
    out = pl.pallas_call(
        _body,
        out_shape=jax.ShapeDtypeStruct((1, 1), jnp.float32),
        out_specs=pl.BlockSpec(memory_space=pltpu.SMEM),
        scratch_shapes=[pltpu.VMEM((N, N), jnp.float32)],
    )(lx, ly, c0, c1, labf)
    return out[0, 0]


# speedup vs baseline: 1.8510x; 1.8510x over previous
import jax
import jax.numpy as jnp
from jax.experimental import pallas as pl
from jax.experimental.pallas import tpu as pltpu

RF_CENTERS = 96.5
TH_C = 0.6
IMG_H = 193.0
TH2 = 12.0 * IMG_H
MARGIN = 48.5
BIG = 3.0e8
N = 128
K = 1024
NROW = 16
NCOL = 1024


def _body(lx_ref, ly_ref, c0_ref, c1_ref, lab_ref, labxt_ref, labyt_ref,
          out_ref, min2_ref):
    f32 = jnp.float32

    gx = lx_ref[...] * 10.0
    gy = ly_ref[...] * 10.0
    xmn, xmx = jnp.min(gx), jnp.max(gx)
    ymn, ymx = jnp.min(gy), jnp.max(gy)

    pxt = labxt_ref[...] - RF_CENTERS
    pyt = labyt_ref[...] - RF_CENTERS
    mt = ((pxt >= xmn - MARGIN) & (pxt <= xmx + MARGIN)
          & (pyt >= ymn - MARGIN) & (pyt <= ymx + MARGIN))
    mt32 = mt.astype(f32)

    ii = jax.lax.broadcasted_iota(jnp.int32, (N, N), 0)
    jj = jax.lax.broadcasted_iota(jnp.int32, (N, N), 1)
    tri_low_incl = (jj <= ii).astype(f32)
    colcs = jax.lax.dot_general(
        tri_low_incl, mt32, (((1,), (0,)), ((), ())),
        preferred_element_type=f32)
    coltot = colcs[N - 1:N, :]
    c8 = jax.lax.broadcasted_iota(jnp.int32, (8, 8), 0)
    s8 = jax.lax.broadcasted_iota(jnp.int32, (8, 8), 1)
    t8_strict = (c8 < s8).astype(f32)
    colpref = jax.lax.dot_general(
        coltot, t8_strict, (((1,), (0,)), ((), ())),
        preferred_element_type=f32)
    rankt = colcs + colpref
    count = colpref[0, 7] + coltot[0, 7]

    labf = lab_ref[...]
    px = labf[:, 0:1] - RF_CENTERS
    py = labf[:, 1:2] - RF_CENTERS
    pp = px * px + py * py
    p4 = jnp.concatenate(
        [px * -2.0, py * -2.0, pp, jnp.ones((K, 1), f32)], axis=1)
    p4_chunks = [jax.lax.slice(p4, (c * N, 0), ((c + 1) * N, 4))
                 for c in range(8)]

    min2_ref[...] = jnp.full((NROW, NCOL), BIG, f32)

    lane = jax.lax.broadcasted_iota(jnp.int32, (N, N), 1).astype(f32)
    slot_col = jax.lax.broadcasted_iota(jnp.int32, (N, 1), 0).astype(f32)
    col4 = jax.lax.broadcasted_iota(jnp.int32, (1, 4), 1)
    bigrow = jnp.where(col4 == 2, BIG, 0.0).astype(f32)

    count_i = count.astype(jnp.int32)
    trips = jnp.maximum((count_i + N - 1) // N, 1)

    def trip(m, _):
        base = m.astype(f32) * N
        cand4 = jnp.zeros((N, 4), f32)
        for c in range(8):
            onehot = (((rankt[:, c:c + 1] - base) == lane[:N, :] + 1.0)
                      & mt[:, c:c + 1]).astype(f32)
            cand4 = cand4 + jax.lax.dot_general(
                onehot, p4_chunks[c], (((0,), (0,)), ((), ())),
                preferred_element_type=f32)
        guard = (slot_col >= (count - base)).astype(f32)
        cand4 = cand4 + guard * bigrow

        def loc(t, _):
            gxt = lx_ref[pl.ds(t, 1), :] * 10.0
            gyt = ly_ref[pl.ds(t, 1), :] * 10.0
            qqt = gxt * gxt + gyt * gyt
            q4 = jnp.concatenate(
                [gxt, gyt, jnp.ones((1, NCOL), f32), qqt], axis=0)
            d = jax.lax.dot_general(
                cand4, q4, (((1,), (0,)), ((), ())),
                preferred_element_type=f32)
            mrow = jnp.min(d, axis=0, keepdims=True)
            min2_ref[pl.ds(t, 1), :] = jnp.minimum(
                min2_ref[pl.ds(t, 1), :], mrow)
            return 0

        jax.lax.fori_loop(0, NROW, loc, 0, unroll=4)
        return 0

    jax.lax.fori_loop(0, trips, trip, 0)

    min2 = min2_ref[...].reshape(N, N)
    c0 = c0_ref[...]
    c1 = c1_ref[...]

    near = min2 < TH2
    far = min2 > TH2
    pos = c1 > TH_C
    neg = c0 > TH_C
    tp = pos & near
    fp = pos & far
    tn = neg & far
    fn = neg & near

    mx = jnp.maximum(c0, c1)
    lse = mx + jnp.log(jnp.exp(c0 - mx) + jnp.exp(c1 - mx))
    v0 = lse - c0
    v1 = lse - c1

    tri_incl = (ii <= jj).astype(f32)
    tri_strict = (jj < ii).astype(f32)

    def mask_stats(mask):
        m32 = mask.astype(f32)
        csum = jax.lax.dot_general(
            m32, tri_incl, (((1,), (0,)), ((), ())),
            preferred_element_type=f32)
        tot = csum[:, N - 1:N]
        pref = jax.lax.dot_general(
            tri_strict, tot, (((1,), (0,)), ((), ())),
            preferred_element_type=f32)
        rank_incl = pref + csum
        cnt = pref[N - 1, 0] + tot[N - 1, 0]
        return m32, rank_incl, cnt

    tp32, tp_rank, tp_n = mask_stats(tp)
    fp32, fp_rank, fp_n = mask_stats(fp)
    tn32, tn_rank, tn_n = mask_stats(tn)
    fn32, fn_rank, fn_n = mask_stats(fn)

    min_n = jnp.minimum(jnp.minimum(tp_n, fp_n), jnp.minimum(tn_n, fn_n))
    min_n = jnp.where(min_n == 0.0, 10.0, min_n)

    def term(m32, rank_incl, cnt, v):
        incl = (m32 > 0.0) & (rank_incl <= min_n)
        s = jnp.sum(jnp.where(incl, v, 0.0))
        n = jnp.minimum(cnt, min_n)
        return jnp.where(cnt > 0.0, s / jnp.maximum(n, 1.0), 0.0)

    loss = (term(tp32, tp_rank, tp_n, v1)
            + term(fp32, fp_rank, fp_n, v0)
            + term(tn32, tn_rank, tn_n, v0)
            + term(fn32, fn_rank, fn_n, v1))
    out_ref[0, 0] = loss


@jax.jit
def kernel(outputs_l, outputs_c, labels):
    lx = outputs_l[0, 0].reshape(NROW, NCOL)
    ly = outputs_l[0, 1].reshape(NROW, NCOL)
    c0 = outputs_c[0, 0]
    c1 = outputs_c[0, 1]
    labf = labels[0].astype(jnp.float32)
    labxt = labf[:, 0].reshape(8, N).T
    labyt = labf[:, 1].reshape(8, N).T
    out = pl.pallas_call(
        _body,
        out_shape=jax.ShapeDtypeStruct((1, 1), jnp.float32),
        out_specs=pl.BlockSpec(memory_space=pltpu.SMEM),
        scratch_shapes=[pltpu.VMEM((NROW, NCOL), jnp.float32)],
    )(lx, ly, c0, c1, labf, labxt, labyt)
    return out[0, 0]


# speedup vs baseline: 2.5372x; 1.3707x over previous
import jax
import jax.numpy as jnp
from jax.experimental import pallas as pl
from jax.experimental.pallas import tpu as pltpu

RF_CENTERS = 96.5
TH_C = 0.6
IMG_H = 193.0
TH2 = 12.0 * IMG_H
MARGIN = 48.5
BIG = 3.0e8
N = 128
K = 1024
NROW = 16
NCOL = 1024


def _body(ol_ref, oc_ref, lab_ref, out_ref, gxs_ref, gys_ref, min2_ref):
    f32 = jnp.float32

    gx = ol_ref[0, 0].reshape(NROW, NCOL) * 10.0
    gy = ol_ref[0, 1].reshape(NROW, NCOL) * 10.0
    gxs_ref[...] = gx
    gys_ref[...] = gy
    xmn, xmx = jnp.min(gx), jnp.max(gx)
    ymn, ymx = jnp.min(gy), jnp.max(gy)

    labf = lab_ref[0].astype(f32)
    px = labf[:, 0:1] - RF_CENTERS
    py = labf[:, 1:2] - RF_CENTERS
    px8 = px.reshape(8, N)
    py8 = py.reshape(8, N)
    m8 = ((px8 >= xmn - MARGIN) & (px8 <= xmx + MARGIN)
          & (py8 >= ymn - MARGIN) & (py8 <= ymx + MARGIN))
    m832 = m8.astype(f32)

    ii = jax.lax.broadcasted_iota(jnp.int32, (N, N), 0)
    jj = jax.lax.broadcasted_iota(jnp.int32, (N, N), 1)
    tri_incl = (ii <= jj).astype(f32)
    tri_strict = (jj < ii).astype(f32)
    i8 = jax.lax.broadcasted_iota(jnp.int32, (8, 8), 0)
    j8 = jax.lax.broadcasted_iota(jnp.int32, (8, 8), 1)
    t8_strict = (j8 < i8).astype(f32)

    csum8 = jax.lax.dot_general(
        m832, tri_incl, (((1,), (0,)), ((), ())),
        preferred_element_type=f32)
    tot8 = csum8[:, N - 1:N]
    pref8 = jax.lax.dot_general(
        t8_strict, tot8, (((1,), (0,)), ((), ())),
        preferred_element_type=f32)
    rank8 = pref8 + csum8
    count = pref8[7, 0] + tot8[7, 0]

    rankt = rank8.T
    mt = m832.T > 0.0

    pp = px * px + py * py
    p4 = jnp.concatenate(
        [px * -2.0, py * -2.0, pp, jnp.ones((K, 1), f32)], axis=1)
    p4_chunks = [jax.lax.slice(p4, (c * N, 0), ((c + 1) * N, 4))
                 for c in range(8)]

    lane = jj.astype(f32)
    slot_col = ii[:, 0:1].astype(f32)
    col4 = jax.lax.broadcasted_iota(jnp.int32, (1, 4), 1)
    bigrow = jnp.where(col4 == 2, BIG, 0.0).astype(f32)

    count_i = count.astype(jnp.int32)
    trips = jnp.maximum((count_i + N - 1) // N, 1)
    min2_ref[...] = jnp.full((NROW, NCOL), BIG, f32)

    def trip(m, acc):
        base = m.astype(f32) * N
        cand4 = jnp.zeros((N, 4), f32)
        for c in range(8):
            onehot = (((rankt[:, c:c + 1] - base) == lane + 1.0)
                      & mt[:, c:c + 1]).astype(f32)
            cand4 = cand4 + jax.lax.dot_general(
                onehot, p4_chunks[c], (((0,), (0,)), ((), ())),
                preferred_element_type=f32)
        guard = (slot_col >= (count - base)).astype(f32)
        cand4 = cand4 + guard * bigrow

        def loc(t, acc_t):
            gxt = gxs_ref[pl.ds(t, 1), :]
            gyt = gys_ref[pl.ds(t, 1), :]
            qqt = gxt * gxt + gyt * gyt
            q4 = jnp.concatenate(
                [gxt, gyt, jnp.ones((1, NCOL), f32), qqt], axis=0)
            d = jax.lax.dot_general(
                cand4, q4, (((1,), (0,)), ((), ())),
                preferred_element_type=f32)
            mrow = jnp.min(d, axis=0, keepdims=True)
            min2_ref[pl.ds(t, 1), :] = jnp.minimum(
                min2_ref[pl.ds(t, 1), :], mrow)
            return acc_t

        return jax.lax.fori_loop(0, NROW, loc, acc, unroll=4)

    jax.lax.fori_loop(0, trips, trip, 0)

    min2 = min2_ref[...].reshape(N, N)
    c0 = oc_ref[0, 0]
    c1 = oc_ref[0, 1]

    near = min2 < TH2
    far = min2 > TH2
    pos = c1 > TH_C
    neg = c0 > TH_C
    tp = pos & near
    fp = pos & far
    tn = neg & far
    fn = neg & near

    mx = jnp.maximum(c0, c1)
    lse = mx + jnp.log(jnp.exp(c0 - mx) + jnp.exp(c1 - mx))
    v0 = lse - c0
    v1 = lse - c1

    def mask_stats(mask):
        m32 = mask.astype(f32)
        csum = jax.lax.dot_general(
            m32, tri_incl, (((1,), (0,)), ((), ())),
            preferred_element_type=f32)
        tot = csum[:, N - 1:N]
        pref = jax.lax.dot_general(
            tri_strict, tot, (((1,), (0,)), ((), ())),
            preferred_element_type=f32)
        rank_incl = pref + csum
        cnt = pref[N - 1, 0] + tot[N - 1, 0]
        return m32, rank_incl, cnt

    tp32, tp_rank, tp_n = mask_stats(tp)
    fp32, fp_rank, fp_n = mask_stats(fp)
    tn32, tn_rank, tn_n = mask_stats(tn)
    fn32, fn_rank, fn_n = mask_stats(fn)

    min_n = jnp.minimum(jnp.minimum(tp_n, fp_n), jnp.minimum(tn_n, fn_n))
    min_n = jnp.where(min_n == 0.0, 10.0, min_n)

    def term(m32, rank_incl, cnt, v):
        incl = (m32 > 0.0) & (rank_incl <= min_n)
        s = jnp.sum(jnp.where(incl, v, 0.0))
        n = jnp.minimum(cnt, min_n)
        return jnp.where(cnt > 0.0, s / jnp.maximum(n, 1.0), 0.0)

    loss = (term(tp32, tp_rank, tp_n, v1)
            + term(fp32, fp_rank, fp_n, v0)
            + term(tn32, tn_rank, tn_n, v0)
            + term(fn32, fn_rank, fn_n, v1))
    out_ref[0, 0] = loss


@jax.jit
def kernel(outputs_l, outputs_c, labels):
    out = pl.pallas_call(
        _body,
        out_shape=jax.ShapeDtypeStruct((1, 1), jnp.float32),
        out_specs=pl.BlockSpec(memory_space=pltpu.SMEM),
        scratch_shapes=[pltpu.VMEM((NROW, NCOL), jnp.float32),
                        pltpu.VMEM((NROW, NCOL), jnp.float32),
                        pltpu.VMEM((NROW, NCOL), jnp.float32)],
    )(outputs_l, outputs_c, labels)
    return out[0, 0]


# speedup vs baseline: 2.9873x; 1.1774x over previous
import jax
import jax.numpy as jnp
from jax.experimental import pallas as pl
from jax.experimental.pallas import tpu as pltpu

RF_CENTERS = 96.5
TH_C = 0.6
IMG_H = 193.0
TH2 = 12.0 * IMG_H
MARGIN = 48.5
BIG = 3.0e8
N = 128
K = 1024
NSLOT = 32
NROW = 4
NCOL = 4096


def _body(ol_ref, oc_ref, lab_ref, out_ref, q4s_ref):
    f32 = jnp.float32

    gx = ol_ref[0, 0].reshape(NROW, NCOL) * 10.0
    gy = ol_ref[0, 1].reshape(NROW, NCOL) * 10.0
    xmn, xmx = jnp.min(gx), jnp.max(gx)
    ymn, ymx = jnp.min(gy), jnp.max(gy)
    qq = gx * gx + gy * gy
    one_row = jnp.ones((1, NCOL), f32)
    for t in range(NROW):
        q4s_ref[4 * t:4 * t + 4, :] = jnp.concatenate(
            [gx[t:t + 1], gy[t:t + 1], one_row, qq[t:t + 1]], axis=0)

    labf = lab_ref[0].astype(f32)
    px = labf[:, 0:1] - RF_CENTERS
    py = labf[:, 1:2] - RF_CENTERS
    px8 = px.reshape(8, N)
    py8 = py.reshape(8, N)
    m8 = ((px8 >= xmn - MARGIN) & (px8 <= xmx + MARGIN)
          & (py8 >= ymn - MARGIN) & (py8 <= ymx + MARGIN))
    m832 = m8.astype(f32)

    ii = jax.lax.broadcasted_iota(jnp.int32, (N, N), 0)
    jj = jax.lax.broadcasted_iota(jnp.int32, (N, N), 1)
    tri_incl = (ii <= jj).astype(f32)
    tri_strict = (jj < ii).astype(f32)
    i8 = jax.lax.broadcasted_iota(jnp.int32, (8, 8), 0)
    j8 = jax.lax.broadcasted_iota(jnp.int32, (8, 8), 1)
    t8_strict = (j8 < i8).astype(f32)

    csum8 = jax.lax.dot_general(
        m832, tri_incl, (((1,), (0,)), ((), ())),
        preferred_element_type=f32)
    tot8 = csum8[:, N - 1:N]
    pref8 = jax.lax.dot_general(
        t8_strict, tot8, (((1,), (0,)), ((), ())),
        preferred_element_type=f32)
    rank8 = pref8 + csum8
    count = pref8[7, 0] + tot8[7, 0]

    rankt = rank8.T
    mt = m832.T > 0.0

    pp = px * px + py * py
    p4 = jnp.concatenate(
        [px * -2.0, py * -2.0, pp, jnp.ones((K, 1), f32)], axis=1)
    p4_chunks = [jax.lax.slice(p4, (c * N, 0), ((c + 1) * N, 4))
                 for c in range(8)]

    lane32 = jax.lax.broadcasted_iota(
        jnp.int32, (N, NSLOT), 1).astype(f32)
    slot_col = jax.lax.broadcasted_iota(
        jnp.int32, (NSLOT, 1), 0).astype(f32)
    col4 = jax.lax.broadcasted_iota(jnp.int32, (1, 4), 1)
    bigrow = jnp.where(col4 == 2, BIG, 0.0).astype(f32)

    count_i = count.astype(jnp.int32)
    trips = jnp.maximum((count_i + NSLOT - 1) // NSLOT, 1)

    def trip(m, acc):
        base = m.astype(f32) * NSLOT
        cand4 = jnp.zeros((NSLOT, 4), f32)
        for c in range(8):
            onehot = (((rankt[:, c:c + 1] - base) == lane32 + 1.0)
                      & mt[:, c:c + 1]).astype(f32)
            cand4 = cand4 + jax.lax.dot_general(
                onehot, p4_chunks[c], (((0,), (0,)), ((), ())),
                preferred_element_type=f32)
        guard = (slot_col >= (count - base)).astype(f32)
        cand4 = cand4 + guard * bigrow

        rows = []
        for t in range(NROW):
            d = jax.lax.dot_general(
                cand4, q4s_ref[4 * t:4 * t + 4, :], (((1,), (0,)), ((), ())),
                preferred_element_type=f32)
            rows.append(jnp.min(d, axis=0, keepdims=True))
        return jnp.minimum(acc, jnp.concatenate(rows, axis=0))

    min2_flat = jax.lax.fori_loop(
        0, trips, trip, jnp.full((NROW, NCOL), BIG, f32))

    min2 = min2_flat.reshape(N, N)
    c0 = oc_ref[0, 0]
    c1 = oc_ref[0, 1]

    near = min2 < TH2
    far = min2 > TH2
    pos = c1 > TH_C
    neg = c0 > TH_C
    tp = pos & near
    fp = pos & far
    tn = neg & far
    fn = neg & near

    mx = jnp.maximum(c0, c1)
    lse = mx + jnp.log(jnp.exp(c0 - mx) + jnp.exp(c1 - mx))
    v0 = lse - c0
    v1 = lse - c1

    def mask_stats(mask):
        m32 = mask.astype(f32)
        csum = jax.lax.dot_general(
            m32, tri_incl, (((1,), (0,)), ((), ())),
            preferred_element_type=f32)
        tot = csum[:, N - 1:N]
        pref = jax.lax.dot_general(
            tri_strict, tot, (((1,), (0,)), ((), ())),
            preferred_element_type=f32)
        rank_incl = pref + csum
        cnt = pref[N - 1, 0] + tot[N - 1, 0]
        return m32, rank_incl, cnt

    tp32, tp_rank, tp_n = mask_stats(tp)
    fp32, fp_rank, fp_n = mask_stats(fp)
    tn32, tn_rank, tn_n = mask_stats(tn)
    fn32, fn_rank, fn_n = mask_stats(fn)

    min_n = jnp.minimum(jnp.minimum(tp_n, fp_n), jnp.minimum(tn_n, fn_n))
    min_n = jnp.where(min_n == 0.0, 10.0, min_n)

    def term(m32, rank_incl, cnt, v):
        incl = (m32 > 0.0) & (rank_incl <= min_n)
        s = jnp.sum(jnp.where(incl, v, 0.0))
        n = jnp.minimum(cnt, min_n)
        return jnp.where(cnt > 0.0, s / jnp.maximum(n, 1.0), 0.0)

    loss = (term(tp32, tp_rank, tp_n, v1)
            + term(fp32, fp_rank, fp_n, v0)
            + term(tn32, tn_rank, tn_n, v0)
            + term(fn32, fn_rank, fn_n, v1))
    out_ref[0, 0] = loss


@jax.jit
def kernel(outputs_l, outputs_c, labels):
    out = pl.pallas_call(
        _body,
        out_shape=jax.ShapeDtypeStruct((1, 1), jnp.float32),
        out_specs=pl.BlockSpec(memory_space=pltpu.SMEM),
        scratch_shapes=[pltpu.VMEM((4 * NROW, NCOL), jnp.float32)],
    )(outputs_l, outputs_c, labels)
    return out[0, 0]
